# baseline (speedup 1.0000x reference)
import functools

import jax
import jax.numpy as jnp
from jax import lax
from jax.experimental import pallas as pl
from jax.experimental.pallas import tpu as pltpu
from jax.experimental.pallas import tpu_sc as plsc

NF = 4000
NC = 10000
H = 128
L = 2
G = 4

NCORE = 2
NSUB = 16
CHUNK = 128
KB = 16

NC_PAD = 10112
NF_PAD = 4096
E1_PAD = 327680
E2_PAD = 65536
CPW1_C0 = 32
CPW1_C1 = 128
CPW2_C0 = 0
CPW2_C1 = 32


def _scale_chunk(g, w_v, gbuf):
    def outer(i16, c):
        wv = w_v[g, pl.ds(i16 * 16, 16)]
        for e in range(16):
            s = wv[e]
            i = i16 * 16 + e
            for j in range(H // 16):
                gbuf[i, pl.ds(j * 16, 16)] = gbuf[i, pl.ds(j * 16, 16)] * s
        return c

    lax.fori_loop(0, CHUNK // 16, outer, 0, unroll=False)


def _seg_pipeline(h_hbm, src_hbm, dst_hbm, w_hbm, acc,
                  src_v, dst_v, w_v, bufs, sems, start_chunk, cpw):
    b0, b1 = bufs
    s0, s1 = sems
    nb = cpw // KB

    def start_gather(g, gbuf, sem):
        pltpu.async_copy(h_hbm.at[src_v.at[g]], gbuf, sem)

    def wait_gather(gbuf, sem):
        pltpu.make_async_copy(h_hbm.at[src_v.at[0]], gbuf, sem).wait()

    def block_body(b, carry):
        pltpu.sync_copy(src_hbm.at[pl.ds(start_chunk + b * KB, KB)], src_v)
        pltpu.sync_copy(dst_hbm.at[pl.ds(start_chunk + b * KB, KB)], dst_v)
        pltpu.sync_copy(w_hbm.at[pl.ds(start_chunk + b * KB, KB)], w_v)
        start_gather(0, b0, s0)
        start_gather(1, b1, s1)

        def pair_body(k, c2):
            g = 2 * k
            wait_gather(b0, s0)
            _scale_chunk(g, w_v, b0)
            pltpu.sync_copy(b0, acc.at[dst_v.at[g]], add=True)
            start_gather(g + 2, b0, s0)
            wait_gather(b1, s1)
            _scale_chunk(g + 1, w_v, b1)
            pltpu.sync_copy(b1, acc.at[dst_v.at[g + 1]], add=True)
            start_gather(g + 3, b1, s1)
            return c2

        lax.fori_loop(0, KB // 2 - 1, pair_body, 0, unroll=False)
        g = KB - 2
        wait_gather(b0, s0)
        _scale_chunk(g, w_v, b0)
        pltpu.sync_copy(b0, acc.at[dst_v.at[g]], add=True)
        wait_gather(b1, s1)
        _scale_chunk(g + 1, w_v, b1)
        pltpu.sync_copy(b1, acc.at[dst_v.at[g + 1]], add=True)
        return carry

    lax.fori_loop(0, nb, block_body, 0, unroll=False)


_SEG_SCRATCH = [
    pltpu.VMEM((KB, CHUNK), jnp.int32),
    pltpu.VMEM((KB, CHUNK), jnp.int32),
    pltpu.VMEM((KB, CHUNK), jnp.float32),
    pltpu.VMEM((CHUNK, H), jnp.float32),
    pltpu.VMEM((CHUNK, H), jnp.float32),
    pltpu.VMEM((8, 16), jnp.float32),
    pltpu.SemaphoreType.DMA,
    pltpu.SemaphoreType.DMA,
]


def _make_seg_all():
    cpw1 = E1_PAD // (CHUNK * NCORE * NSUB)
    cpw2 = E2_PAD // (CHUNK * NCORE * NSUB)
    rps_c = NC_PAD // NSUB
    rps_f = NF_PAD // NSUB
    mesh = plsc.VectorSubcoreMesh(
        core_axis_name="c", subcore_axis_name="s",
        num_cores=NCORE, num_subcores=NSUB)

    @functools.partial(
        pl.kernel,
        out_type=(jax.ShapeDtypeStruct((NCORE, NC_PAD, H), jnp.float32),
                  jax.ShapeDtypeStruct((NCORE, NF_PAD, H), jnp.float32),
                  jax.ShapeDtypeStruct((NCORE, NF_PAD, H), jnp.float32)),
        mesh=mesh,
        scratch_types=_SEG_SCRATCH[:5]
        + [pltpu.VMEM_SHARED((NC_PAD, H), jnp.float32)]
        + _SEG_SCRATCH[5:],
    )
    def seg3(hf_hbm, hc_hbm,
             src1_hbm, dst1_hbm, w1_hbm,
             src2_hbm, dst2_hbm, w2_hbm,
             src3_hbm, dst3_hbm, w3_hbm, zeros_hbm,
             outc_hbm, outf1_hbm, outf2_hbm,
             src_v, dst_v, w_v, b0, b1, acc, tok_v, s0, s1):
        cid = lax.axis_index("c")
        sid = lax.axis_index("s")
        wid = sid * NCORE + cid

        def zero_acc(rps):
            pltpu.sync_copy(zeros_hbm.at[pl.ds(sid * rps, rps)],
                            acc.at[pl.ds(sid * rps, rps)])

        def writeout(out_hbm, rps):
            pltpu.sync_copy(acc.at[pl.ds(sid * rps, rps)],
                            out_hbm.at[cid, pl.ds(sid * rps, rps)])

        def pipeline(h_hbm, src_hbm, dst_hbm, w_hbm, cpw0, cpw1c):
            if cpw0 > 0:
                @pl.when(cid == 0)
                def _():
                    _seg_pipeline(h_hbm, src_hbm, dst_hbm, w_hbm, acc,
                                  src_v, dst_v, w_v, (b0, b1), (s0, s1),
                                  sid * cpw0, cpw0)
            if cpw1c > 0:
                @pl.when(cid == 1)
                def _():
                    _seg_pipeline(h_hbm, src_hbm, dst_hbm, w_hbm, acc,
                                  src_v, dst_v, w_v, (b0, b1), (s0, s1),
                                  NSUB * cpw0 + sid * cpw1c, cpw1c)

        zero_acc(rps_c)
        plsc.subcore_barrier()
        pipeline(hf_hbm, src1_hbm, dst1_hbm, w1_hbm, CPW1_C0, CPW1_C1)
        plsc.subcore_barrier()
        writeout(outc_hbm, rps_c)
        plsc.subcore_barrier()
        zero_acc(rps_f)
        plsc.subcore_barrier()
        pipeline(hc_hbm, src2_hbm, dst2_hbm, w2_hbm, CPW1_C0, CPW1_C1)
        plsc.subcore_barrier()
        writeout(outf1_hbm, rps_f)
        plsc.subcore_barrier()
        zero_acc(rps_f)
        plsc.subcore_barrier()
        pipeline(hf_hbm, src3_hbm, dst3_hbm, w3_hbm, CPW2_C0, CPW2_C1)
        plsc.subcore_barrier()
        writeout(outf2_hbm, rps_f)

    return seg3


def _make_deg():
    cpw1 = E1_PAD // (CHUNK * NCORE * NSUB)
    cpw2 = E2_PAD // (CHUNK * NCORE * NSUB)
    rps_c = NC_PAD // NSUB
    rps_f = NF_PAD // NSUB
    mesh = plsc.VectorSubcoreMesh(
        core_axis_name="c", subcore_axis_name="s",
        num_cores=NCORE, num_subcores=NSUB)

    @functools.partial(
        pl.kernel,
        out_type=(jax.ShapeDtypeStruct((NCORE, NC_PAD, 16), jnp.float32),
                  jax.ShapeDtypeStruct((NCORE, NF_PAD, 16), jnp.float32),
                  jax.ShapeDtypeStruct((NCORE, NF_PAD, 16), jnp.float32)),
        mesh=mesh,
        scratch_types=[
            pltpu.VMEM((cpw1, CHUNK), jnp.int32),
            pltpu.VMEM((CHUNK, 16), jnp.float32),
            pltpu.VMEM_SHARED((NC_PAD, 16), jnp.float32),
        ],
    )
    def deg(dst1_hbm, dst2_hbm, dst3_hbm, ones_hbm, zeros_hbm,
            out1_hbm, out2_hbm, out3_hbm,
            dst_v, ones_v, acc):
        cid = lax.axis_index("c")
        sid = lax.axis_index("s")
        wid = sid * NCORE + cid

        pltpu.sync_copy(ones_hbm, ones_v)

        def zero_acc(rps):
            pltpu.sync_copy(zeros_hbm.at[pl.ds(sid * rps, rps)],
                            acc.at[pl.ds(sid * rps, rps)])

        def run_pass(dst_hbm, cpw):
            pltpu.sync_copy(dst_hbm.at[pl.ds(wid * cpw, cpw)],
                            dst_v.at[pl.ds(0, cpw)])

            def chunk(g, carry):
                pltpu.sync_copy(ones_v, acc.at[dst_v.at[g]], add=True)
                return carry

            lax.fori_loop(0, cpw, chunk, 0, unroll=False)

        def writeout(out_hbm, rps):
            pltpu.sync_copy(acc.at[pl.ds(sid * rps, rps)],
                            out_hbm.at[cid, pl.ds(sid * rps, rps)])

        zero_acc(rps_c)
        plsc.subcore_barrier()
        run_pass(dst1_hbm, cpw1)
        plsc.subcore_barrier()
        writeout(out1_hbm, rps_c)
        plsc.subcore_barrier()
        zero_acc(rps_f)
        plsc.subcore_barrier()
        run_pass(dst2_hbm, cpw1)
        plsc.subcore_barrier()
        writeout(out2_hbm, rps_f)
        plsc.subcore_barrier()
        zero_acc(rps_f)
        plsc.subcore_barrier()
        run_pass(dst3_hbm, cpw2)
        plsc.subcore_barrier()
        writeout(out3_hbm, rps_f)

    return deg


def _gelu(x):
    return 0.5 * x * (1.0 + lax.erf(x * 0.7071067811865476))


def _group_masks():
    gpc = H // G
    Mg = (lax.broadcasted_iota(jnp.int32, (H, G), 0) // gpc
          == lax.broadcasted_iota(jnp.int32, (H, G), 1)).astype(jnp.float32)
    MgT = (lax.broadcasted_iota(jnp.int32, (G, H), 1) // gpc
           == lax.broadcasted_iota(jnp.int32, (G, H), 0)).astype(jnp.float32)
    return Mg / gpc, MgT


def _gn_apply(x, g, b):
    Mg, MgT = _group_masks()
    m = lax.dot(x, Mg, preferred_element_type=jnp.float32)
    mb = lax.dot(m, MgT, preferred_element_type=jnp.float32)
    xc = x - mb
    v = lax.dot(xc * xc, Mg, preferred_element_type=jnp.float32)
    vb = lax.dot(v, MgT, preferred_element_type=jnp.float32)
    return xc * lax.rsqrt(vb + 1e-5) * g + b


def _dot(a, b):
    return lax.dot(a, b, preferred_element_type=jnp.float32)


def _init_f_body(x_ref, W_ref, b_ref, g_ref, bn_ref, o_ref):
    x = x_ref[...]
    x = jnp.where(x >= 0, x, 0.01 * x)
    y = _dot(x, W_ref[...]) + b_ref[...]
    o_ref[...] = _gn_apply(_gelu(y), g_ref[...], bn_ref[...])


def _init_c_body(ids_ref, ec_ref, W_ref, b_ref, g_ref, bn_ref, o_ref):
    ids = ids_ref[...]
    ec = ec_ref[...]
    x = jnp.where(ids == 0, ec[0:1, :], ec[1:2, :])
    x = jnp.where(x >= 0, x, 0.01 * x)
    y = _dot(x, W_ref[...]) + b_ref[...]
    o_ref[...] = _gn_apply(_gelu(y), g_ref[...], bn_ref[...])


def _cell_body(hc_ref, s_ref, r_ref, Ws_ref, Wn_ref, b_ref, g_ref, bn_ref, o_ref):
    s = (s_ref[0] + s_ref[1]) * r_ref[...]
    m = _dot(hc_ref[...], Ws_ref[...]) + _dot(s, Wn_ref[...]) + b_ref[...]
    o_ref[...] = _gelu(_gn_apply(m, g_ref[...], bn_ref[...]))


def _feat_body(hf_ref, s1_ref, r1_ref, s2_ref, r2_ref,
               Ws1_ref, Wn1_ref, b1_ref, Ws2_ref, Wn2_ref, b2_ref,
               g1_ref, bn1_ref, g2_ref, bn2_ref, o_ref):
    hf = hf_ref[...]
    s1 = (s1_ref[0] + s1_ref[1]) * r1_ref[...]
    s2 = (s2_ref[0] + s2_ref[1]) * r2_ref[...]
    m1 = _dot(hf, Ws1_ref[...]) + _dot(s1, Wn1_ref[...]) + b1_ref[...]
    m2 = _dot(hf, Ws2_ref[...]) + _dot(s2, Wn2_ref[...]) + b2_ref[...]
    h1 = _gn_apply(m1, g1_ref[...], bn1_ref[...])
    h2 = _gn_apply(m2, g2_ref[...], bn2_ref[...])
    o_ref[...] = _gelu(0.5 * h1 + 0.5 * h2)


def _row_spec(bn):
    return pl.BlockSpec((bn, H), lambda i: (i, 0))


def _full_spec(shape):
    return pl.BlockSpec(shape, lambda i: tuple(0 for _ in shape))


def _init_f(x, W, b, g, bn):
    bn_rows = 1000
    return pl.pallas_call(
        _init_f_body,
        grid=(NF // bn_rows,),
        in_specs=[_row_spec(bn_rows), _full_spec((H, H)), _full_spec((1, H)),
                  _full_spec((1, H)), _full_spec((1, H))],
        out_specs=_row_spec(bn_rows),
        out_shape=jax.ShapeDtypeStruct((NF, H), jnp.float32),
    )(x, W, b.reshape(1, H), g.reshape(1, H), bn.reshape(1, H))


def _init_c(ids, ec, W, b, g, bn):
    bn_rows = 1000
    return pl.pallas_call(
        _init_c_body,
        grid=(NC // bn_rows,),
        in_specs=[pl.BlockSpec((bn_rows, 1), lambda i: (i, 0)),
                  _full_spec((2, H)), _full_spec((H, H)), _full_spec((1, H)),
                  _full_spec((1, H)), _full_spec((1, H))],
        out_specs=_row_spec(bn_rows),
        out_shape=jax.ShapeDtypeStruct((NC, H), jnp.float32),
    )(ids.reshape(NC, 1), ec, W, b.reshape(1, H), g.reshape(1, H), bn.reshape(1, H))


def _cell_dense(hc, s, r, Ws, Wn, b, g, bn):
    bn_rows = 1000
    return pl.pallas_call(
        _cell_body,
        grid=(NC // bn_rows,),
        in_specs=[_row_spec(bn_rows),
                  pl.BlockSpec((NCORE, bn_rows, H), lambda i: (0, i, 0)),
                  pl.BlockSpec((bn_rows, 1), lambda i: (i, 0)),
                  _full_spec((H, H)), _full_spec((H, H)), _full_spec((1, H)),
                  _full_spec((1, H)), _full_spec((1, H))],
        out_specs=_row_spec(bn_rows),
        out_shape=jax.ShapeDtypeStruct((NC, H), jnp.float32),
    )(hc, s, r, Ws, Wn, b.reshape(1, H), g.reshape(1, H), bn.reshape(1, H))


def _feat_dense(hf, s1, r1, s2, r2, Ws1, Wn1, b1, Ws2, Wn2, b2, g1, bn1, g2, bn2):
    bn_rows = 1000
    return pl.pallas_call(
        _feat_body,
        grid=(NF // bn_rows,),
        in_specs=[_row_spec(bn_rows),
                  pl.BlockSpec((NCORE, bn_rows, H), lambda i: (0, i, 0)),
                  pl.BlockSpec((bn_rows, 1), lambda i: (i, 0)),
                  pl.BlockSpec((NCORE, bn_rows, H), lambda i: (0, i, 0)),
                  pl.BlockSpec((bn_rows, 1), lambda i: (i, 0)),
                  _full_spec((H, H)), _full_spec((H, H)), _full_spec((1, H)),
                  _full_spec((H, H)), _full_spec((H, H)), _full_spec((1, H)),
                  _full_spec((1, H)), _full_spec((1, H)),
                  _full_spec((1, H)), _full_spec((1, H))],
        out_specs=_row_spec(bn_rows),
        out_shape=jax.ShapeDtypeStruct((NF, H), jnp.float32),
    )(hf, s1, r1, s2, r2, Ws1, Wn1, b1.reshape(1, H), Ws2, Wn2, b2.reshape(1, H),
      g1.reshape(1, H), bn1.reshape(1, H), g2.reshape(1, H), bn2.reshape(1, H))


@functools.lru_cache(None)
def _sc_kernels():
    return {
        "seg_all": _make_seg_all(),
        "deg": _make_deg(),
    }


def _pad_edges(src, dst, w, e_pad, n_dst):
    e = src.shape[0]
    pad = e_pad - e
    shp = (e_pad // CHUNK, CHUNK)
    src_p = jnp.concatenate([src, jnp.zeros((pad,), jnp.int32)]).reshape(shp)
    dst_p = jnp.concatenate([dst, jnp.full((pad,), n_dst, jnp.int32)]).reshape(shp)
    w_p = jnp.concatenate([w, jnp.zeros((pad,), jnp.float32)]).reshape(shp)
    return src_p, dst_p, w_p


def _recip_deg(deg16, n):
    d = deg16[:, :n, 0].sum(0)
    return (1.0 / jnp.clip(d, 1.0)).reshape(n, 1)


def kernel(feat_ids, cell_ids, f2c_src, f2c_dst, f2c_w, c2f_src, c2f_dst, c2f_w,
           pw_src, pw_dst, pw_w, embed_feat, embed_cell, in_lin_W, in_lin_b,
           in_norm_g, in_norm_b, sage_Wn, sage_Ws, sage_b, cn_g, cn_b):
    del feat_ids

    f2c_s, f2c_d, f2c_wp = _pad_edges(f2c_src, f2c_dst, f2c_w, E1_PAD, NC)
    c2f_s, c2f_d, c2f_wp = _pad_edges(c2f_src, c2f_dst, c2f_w, E1_PAD, NF)
    pw_s, pw_d, pw_wp = _pad_edges(pw_src, pw_dst, pw_w, E2_PAD, NF)

    zc = jnp.zeros((NC_PAD, H), jnp.float32)
    zf = jnp.zeros((NF_PAD, H), jnp.float32)
    zc16 = jnp.zeros((NC_PAD, 16), jnp.float32)
    ones16 = jnp.concatenate(
        [jnp.ones((CHUNK, 1), jnp.float32), jnp.zeros((CHUNK, 15), jnp.float32)], axis=1)

    hf = _init_f(embed_feat, in_lin_W[1], in_lin_b[1], in_norm_g[1], in_norm_b[1])
    hc = _init_c(cell_ids, embed_cell, in_lin_W[0], in_lin_b[0], in_norm_g[0], in_norm_b[0])

    sck = _sc_kernels()
    dc16, df116, df216 = sck["deg"](f2c_d, c2f_d, pw_d, ones16, zc16)
    rc = _recip_deg(dc16, NC)
    rf1 = _recip_deg(df116, NF)
    rf2 = _recip_deg(df216, NF)

    cn_g_l = cn_g.reshape(L, 3, H)
    cn_b_l = cn_b.reshape(L, 3, H)

    for i in range(L):
        Ws, Wn, b = sage_Ws[i], sage_Wn[i], sage_b[i]
        g_l, b_l = cn_g_l[i], cn_b_l[i]
        s_c, s_f1, s_f2 = sck["seg_all"](hf, hc,
                                         f2c_s, f2c_d, f2c_wp,
                                         c2f_s, c2f_d, c2f_wp,
                                         pw_s, pw_d, pw_wp, zc)
        hc_new = _cell_dense(hc, s_c[:, :NC], rc, Ws[0], Wn[0],
                             b[0], g_l[0], b_l[0])
        hf_new = _feat_dense(hf, s_f1[:, :NF], rf1, s_f2[:, :NF], rf2,
                             Ws[1], Wn[1], b[1],
                             Ws[2], Wn[2], b[2],
                             g_l[1], b_l[1], g_l[2], b_l[2])
        hf, hc = hf_new, hc_new

    return jnp.concatenate([hf, hc], axis=0)


# speedup vs baseline: 1.3503x; 1.3503x over previous
import functools

import jax
import jax.numpy as jnp
from jax import lax
from jax.experimental import pallas as pl
from jax.experimental.pallas import tpu as pltpu
from jax.experimental.pallas import tpu_sc as plsc

NF = 4000
NC = 10000
H = 128
L = 2
G = 4

NCORE = 2
NSUB = 16
CHUNK = 128
KB = 16

NC_PAD = 10112
NF_PAD = 4096
E1_PAD = 327680
E2_PAD = 65536
CPW1_C0 = 128
CPW1_C1 = 32
CPW2_C0 = 32
CPW2_C1 = 0


def _scale_chunk(g, w_v, gbuf):
    def outer(i16, c):
        wv = w_v[g, pl.ds(i16 * 16, 16)]
        for e in range(16):
            s = wv[e]
            i = i16 * 16 + e
            for j in range(H // 16):
                gbuf[i, pl.ds(j * 16, 16)] = gbuf[i, pl.ds(j * 16, 16)] * s
        return c

    lax.fori_loop(0, CHUNK // 16, outer, 0, unroll=False)


def _seg_pipeline(h_hbm, src_hbm, dst_hbm, w_hbm, acc,
                  src_v, dst_v, w_v, bufs, sems, start_chunk, cpw):
    b0, b1 = bufs
    s0, s1 = sems
    nb = cpw // KB

    def start_gather(g, gbuf, sem):
        pltpu.async_copy(h_hbm.at[src_v.at[g]], gbuf, sem)

    def wait_gather(gbuf, sem):
        pltpu.make_async_copy(h_hbm.at[src_v.at[0]], gbuf, sem).wait()

    def block_body(b, carry):
        pltpu.sync_copy(src_hbm.at[pl.ds(start_chunk + b * KB, KB)], src_v)
        pltpu.sync_copy(dst_hbm.at[pl.ds(start_chunk + b * KB, KB)], dst_v)
        pltpu.sync_copy(w_hbm.at[pl.ds(start_chunk + b * KB, KB)], w_v)
        start_gather(0, b0, s0)
        start_gather(1, b1, s1)

        def pair_body(k, c2):
            g = 2 * k
            wait_gather(b0, s0)
            _scale_chunk(g, w_v, b0)
            pltpu.sync_copy(b0, acc.at[dst_v.at[g]], add=True)
            start_gather(g + 2, b0, s0)
            wait_gather(b1, s1)
            _scale_chunk(g + 1, w_v, b1)
            pltpu.sync_copy(b1, acc.at[dst_v.at[g + 1]], add=True)
            start_gather(g + 3, b1, s1)
            return c2

        lax.fori_loop(0, KB // 2 - 1, pair_body, 0, unroll=False)
        g = KB - 2
        wait_gather(b0, s0)
        _scale_chunk(g, w_v, b0)
        pltpu.sync_copy(b0, acc.at[dst_v.at[g]], add=True)
        wait_gather(b1, s1)
        _scale_chunk(g + 1, w_v, b1)
        pltpu.sync_copy(b1, acc.at[dst_v.at[g + 1]], add=True)
        return carry

    lax.fori_loop(0, nb, block_body, 0, unroll=False)


_SEG_SCRATCH = [
    pltpu.VMEM((KB, CHUNK), jnp.int32),
    pltpu.VMEM((KB, CHUNK), jnp.int32),
    pltpu.VMEM((KB, CHUNK), jnp.float32),
    pltpu.VMEM((CHUNK, H), jnp.float32),
    pltpu.VMEM((CHUNK, H), jnp.float32),
    pltpu.VMEM((8, 16), jnp.float32),
    pltpu.SemaphoreType.DMA,
    pltpu.SemaphoreType.DMA,
]


def _make_seg_all():
    cpw1 = E1_PAD // (CHUNK * NCORE * NSUB)
    cpw2 = E2_PAD // (CHUNK * NCORE * NSUB)
    rps_c = NC_PAD // NSUB
    rps_f = NF_PAD // NSUB
    mesh = plsc.VectorSubcoreMesh(
        core_axis_name="c", subcore_axis_name="s",
        num_cores=NCORE, num_subcores=NSUB)

    @functools.partial(
        pl.kernel,
        out_type=(jax.ShapeDtypeStruct((NCORE, NC_PAD, H), jnp.float32),
                  jax.ShapeDtypeStruct((NCORE, NF_PAD, H), jnp.float32),
                  jax.ShapeDtypeStruct((NCORE, NF_PAD, H), jnp.float32)),
        mesh=mesh,
        scratch_types=_SEG_SCRATCH[:5]
        + [pltpu.VMEM_SHARED((NC_PAD, H), jnp.float32)]
        + _SEG_SCRATCH[5:],
    )
    def seg3(hf_hbm, hc_hbm,
             src1_hbm, dst1_hbm, w1_hbm,
             src2_hbm, dst2_hbm, w2_hbm,
             src3_hbm, dst3_hbm, w3_hbm, zeros_hbm,
             outc_hbm, outf1_hbm, outf2_hbm,
             src_v, dst_v, w_v, b0, b1, acc, tok_v, s0, s1):
        cid = lax.axis_index("c")
        sid = lax.axis_index("s")
        wid = sid * NCORE + cid

        def zero_acc(rps):
            pltpu.sync_copy(zeros_hbm.at[pl.ds(sid * rps, rps)],
                            acc.at[pl.ds(sid * rps, rps)])

        def writeout(out_hbm, rps):
            pltpu.sync_copy(acc.at[pl.ds(sid * rps, rps)],
                            out_hbm.at[cid, pl.ds(sid * rps, rps)])

        def pipeline(h_hbm, src_hbm, dst_hbm, w_hbm, cpw0, cpw1c):
            if cpw0 > 0:
                @pl.when(cid == 0)
                def _():
                    _seg_pipeline(h_hbm, src_hbm, dst_hbm, w_hbm, acc,
                                  src_v, dst_v, w_v, (b0, b1), (s0, s1),
                                  sid * cpw0, cpw0)
            if cpw1c > 0:
                @pl.when(cid == 1)
                def _():
                    _seg_pipeline(h_hbm, src_hbm, dst_hbm, w_hbm, acc,
                                  src_v, dst_v, w_v, (b0, b1), (s0, s1),
                                  NSUB * cpw0 + sid * cpw1c, cpw1c)

        zero_acc(rps_c)
        plsc.subcore_barrier()
        pipeline(hf_hbm, src1_hbm, dst1_hbm, w1_hbm, CPW1_C0, CPW1_C1)
        plsc.subcore_barrier()
        writeout(outc_hbm, rps_c)
        plsc.subcore_barrier()
        zero_acc(rps_f)
        plsc.subcore_barrier()
        pipeline(hc_hbm, src2_hbm, dst2_hbm, w2_hbm, CPW1_C0, CPW1_C1)
        plsc.subcore_barrier()
        writeout(outf1_hbm, rps_f)
        plsc.subcore_barrier()
        zero_acc(rps_f)
        plsc.subcore_barrier()
        pipeline(hf_hbm, src3_hbm, dst3_hbm, w3_hbm, CPW2_C0, CPW2_C1)
        plsc.subcore_barrier()
        writeout(outf2_hbm, rps_f)

    return seg3


def _make_deg():
    cpw1 = E1_PAD // (CHUNK * NCORE * NSUB)
    cpw2 = E2_PAD // (CHUNK * NCORE * NSUB)
    rps_c = NC_PAD // NSUB
    rps_f = NF_PAD // NSUB
    mesh = plsc.VectorSubcoreMesh(
        core_axis_name="c", subcore_axis_name="s",
        num_cores=NCORE, num_subcores=NSUB)

    @functools.partial(
        pl.kernel,
        out_type=(jax.ShapeDtypeStruct((NCORE, NC_PAD, 16), jnp.float32),
                  jax.ShapeDtypeStruct((NCORE, NF_PAD, 16), jnp.float32),
                  jax.ShapeDtypeStruct((NCORE, NF_PAD, 16), jnp.float32)),
        mesh=mesh,
        scratch_types=[
            pltpu.VMEM((cpw1, CHUNK), jnp.int32),
            pltpu.VMEM((CHUNK, 16), jnp.float32),
            pltpu.VMEM_SHARED((NC_PAD, 16), jnp.float32),
        ],
    )
    def deg(dst1_hbm, dst2_hbm, dst3_hbm, ones_hbm, zeros_hbm,
            out1_hbm, out2_hbm, out3_hbm,
            dst_v, ones_v, acc):
        cid = lax.axis_index("c")
        sid = lax.axis_index("s")
        wid = sid * NCORE + cid

        pltpu.sync_copy(ones_hbm, ones_v)

        def zero_acc(rps):
            pltpu.sync_copy(zeros_hbm.at[pl.ds(sid * rps, rps)],
                            acc.at[pl.ds(sid * rps, rps)])

        def run_pass(dst_hbm, cpw):
            pltpu.sync_copy(dst_hbm.at[pl.ds(wid * cpw, cpw)],
                            dst_v.at[pl.ds(0, cpw)])

            def chunk(g, carry):
                pltpu.sync_copy(ones_v, acc.at[dst_v.at[g]], add=True)
                return carry

            lax.fori_loop(0, cpw, chunk, 0, unroll=False)

        def writeout(out_hbm, rps):
            pltpu.sync_copy(acc.at[pl.ds(sid * rps, rps)],
                            out_hbm.at[cid, pl.ds(sid * rps, rps)])

        zero_acc(rps_c)
        plsc.subcore_barrier()
        run_pass(dst1_hbm, cpw1)
        plsc.subcore_barrier()
        writeout(out1_hbm, rps_c)
        plsc.subcore_barrier()
        zero_acc(rps_f)
        plsc.subcore_barrier()
        run_pass(dst2_hbm, cpw1)
        plsc.subcore_barrier()
        writeout(out2_hbm, rps_f)
        plsc.subcore_barrier()
        zero_acc(rps_f)
        plsc.subcore_barrier()
        run_pass(dst3_hbm, cpw2)
        plsc.subcore_barrier()
        writeout(out3_hbm, rps_f)

    return deg


def _gelu(x):
    return 0.5 * x * (1.0 + lax.erf(x * 0.7071067811865476))


def _group_masks():
    gpc = H // G
    Mg = (lax.broadcasted_iota(jnp.int32, (H, G), 0) // gpc
          == lax.broadcasted_iota(jnp.int32, (H, G), 1)).astype(jnp.float32)
    MgT = (lax.broadcasted_iota(jnp.int32, (G, H), 1) // gpc
           == lax.broadcasted_iota(jnp.int32, (G, H), 0)).astype(jnp.float32)
    return Mg / gpc, MgT


def _gn_apply(x, g, b):
    Mg, MgT = _group_masks()
    m = lax.dot(x, Mg, preferred_element_type=jnp.float32)
    mb = lax.dot(m, MgT, preferred_element_type=jnp.float32)
    xc = x - mb
    v = lax.dot(xc * xc, Mg, preferred_element_type=jnp.float32)
    vb = lax.dot(v, MgT, preferred_element_type=jnp.float32)
    return xc * lax.rsqrt(vb + 1e-5) * g + b


def _dot(a, b):
    return lax.dot(a, b, preferred_element_type=jnp.float32)


def _init_f_body(x_ref, W_ref, b_ref, g_ref, bn_ref, o_ref):
    x = x_ref[...]
    x = jnp.where(x >= 0, x, 0.01 * x)
    y = _dot(x, W_ref[...]) + b_ref[...]
    o_ref[...] = _gn_apply(_gelu(y), g_ref[...], bn_ref[...])


def _init_c_body(ids_ref, ec_ref, W_ref, b_ref, g_ref, bn_ref, o_ref):
    ids = ids_ref[...]
    ec = ec_ref[...]
    x = jnp.where(ids == 0, ec[0:1, :], ec[1:2, :])
    x = jnp.where(x >= 0, x, 0.01 * x)
    y = _dot(x, W_ref[...]) + b_ref[...]
    o_ref[...] = _gn_apply(_gelu(y), g_ref[...], bn_ref[...])


def _cell_body(hc_ref, s_ref, r_ref, Ws_ref, Wn_ref, b_ref, g_ref, bn_ref, o_ref):
    s = (s_ref[0] + s_ref[1]) * r_ref[...]
    m = _dot(hc_ref[...], Ws_ref[...]) + _dot(s, Wn_ref[...]) + b_ref[...]
    o_ref[...] = _gelu(_gn_apply(m, g_ref[...], bn_ref[...]))


def _feat_body(hf_ref, s1_ref, r1_ref, s2_ref, r2_ref,
               Ws1_ref, Wn1_ref, b1_ref, Ws2_ref, Wn2_ref, b2_ref,
               g1_ref, bn1_ref, g2_ref, bn2_ref, o_ref):
    hf = hf_ref[...]
    s1 = (s1_ref[0] + s1_ref[1]) * r1_ref[...]
    s2 = (s2_ref[0] + s2_ref[1]) * r2_ref[...]
    m1 = _dot(hf, Ws1_ref[...]) + _dot(s1, Wn1_ref[...]) + b1_ref[...]
    m2 = _dot(hf, Ws2_ref[...]) + _dot(s2, Wn2_ref[...]) + b2_ref[...]
    h1 = _gn_apply(m1, g1_ref[...], bn1_ref[...])
    h2 = _gn_apply(m2, g2_ref[...], bn2_ref[...])
    o_ref[...] = _gelu(0.5 * h1 + 0.5 * h2)


def _row_spec(bn):
    return pl.BlockSpec((bn, H), lambda i: (i, 0))


def _full_spec(shape):
    return pl.BlockSpec(shape, lambda i: tuple(0 for _ in shape))


def _init_f(x, W, b, g, bn):
    bn_rows = 1000
    return pl.pallas_call(
        _init_f_body,
        grid=(NF // bn_rows,),
        in_specs=[_row_spec(bn_rows), _full_spec((H, H)), _full_spec((1, H)),
                  _full_spec((1, H)), _full_spec((1, H))],
        out_specs=_row_spec(bn_rows),
        out_shape=jax.ShapeDtypeStruct((NF, H), jnp.float32),
    )(x, W, b.reshape(1, H), g.reshape(1, H), bn.reshape(1, H))


def _init_c(ids, ec, W, b, g, bn):
    bn_rows = 1000
    return pl.pallas_call(
        _init_c_body,
        grid=(NC // bn_rows,),
        in_specs=[pl.BlockSpec((bn_rows, 1), lambda i: (i, 0)),
                  _full_spec((2, H)), _full_spec((H, H)), _full_spec((1, H)),
                  _full_spec((1, H)), _full_spec((1, H))],
        out_specs=_row_spec(bn_rows),
        out_shape=jax.ShapeDtypeStruct((NC, H), jnp.float32),
    )(ids.reshape(NC, 1), ec, W, b.reshape(1, H), g.reshape(1, H), bn.reshape(1, H))


def _cell_dense(hc, s, r, Ws, Wn, b, g, bn):
    bn_rows = 1000
    return pl.pallas_call(
        _cell_body,
        grid=(NC // bn_rows,),
        in_specs=[_row_spec(bn_rows),
                  pl.BlockSpec((NCORE, bn_rows, H), lambda i: (0, i, 0)),
                  pl.BlockSpec((bn_rows, 1), lambda i: (i, 0)),
                  _full_spec((H, H)), _full_spec((H, H)), _full_spec((1, H)),
                  _full_spec((1, H)), _full_spec((1, H))],
        out_specs=_row_spec(bn_rows),
        out_shape=jax.ShapeDtypeStruct((NC, H), jnp.float32),
    )(hc, s, r, Ws, Wn, b.reshape(1, H), g.reshape(1, H), bn.reshape(1, H))


def _feat_dense(hf, s1, r1, s2, r2, Ws1, Wn1, b1, Ws2, Wn2, b2, g1, bn1, g2, bn2):
    bn_rows = 1000
    return pl.pallas_call(
        _feat_body,
        grid=(NF // bn_rows,),
        in_specs=[_row_spec(bn_rows),
                  pl.BlockSpec((NCORE, bn_rows, H), lambda i: (0, i, 0)),
                  pl.BlockSpec((bn_rows, 1), lambda i: (i, 0)),
                  pl.BlockSpec((NCORE, bn_rows, H), lambda i: (0, i, 0)),
                  pl.BlockSpec((bn_rows, 1), lambda i: (i, 0)),
                  _full_spec((H, H)), _full_spec((H, H)), _full_spec((1, H)),
                  _full_spec((H, H)), _full_spec((H, H)), _full_spec((1, H)),
                  _full_spec((1, H)), _full_spec((1, H)),
                  _full_spec((1, H)), _full_spec((1, H))],
        out_specs=_row_spec(bn_rows),
        out_shape=jax.ShapeDtypeStruct((NF, H), jnp.float32),
    )(hf, s1, r1, s2, r2, Ws1, Wn1, b1.reshape(1, H), Ws2, Wn2, b2.reshape(1, H),
      g1.reshape(1, H), bn1.reshape(1, H), g2.reshape(1, H), bn2.reshape(1, H))


@functools.lru_cache(None)
def _sc_kernels():
    return {
        "seg_all": _make_seg_all(),
        "deg": _make_deg(),
    }


def _pad_edges(src, dst, w, e_pad, n_dst):
    e = src.shape[0]
    pad = e_pad - e
    shp = (e_pad // CHUNK, CHUNK)
    src_p = jnp.concatenate([src, jnp.zeros((pad,), jnp.int32)]).reshape(shp)
    dst_p = jnp.concatenate([dst, jnp.full((pad,), n_dst, jnp.int32)]).reshape(shp)
    w_p = jnp.concatenate([w, jnp.zeros((pad,), jnp.float32)]).reshape(shp)
    return src_p, dst_p, w_p


def _recip_deg(deg16, n):
    d = deg16[:, :n, 0].sum(0)
    return (1.0 / jnp.clip(d, 1.0)).reshape(n, 1)


def kernel(feat_ids, cell_ids, f2c_src, f2c_dst, f2c_w, c2f_src, c2f_dst, c2f_w,
           pw_src, pw_dst, pw_w, embed_feat, embed_cell, in_lin_W, in_lin_b,
           in_norm_g, in_norm_b, sage_Wn, sage_Ws, sage_b, cn_g, cn_b):
    del feat_ids

    f2c_s, f2c_d, f2c_wp = _pad_edges(f2c_src, f2c_dst, f2c_w, E1_PAD, NC)
    c2f_s, c2f_d, c2f_wp = _pad_edges(c2f_src, c2f_dst, c2f_w, E1_PAD, NF)
    pw_s, pw_d, pw_wp = _pad_edges(pw_src, pw_dst, pw_w, E2_PAD, NF)

    zc = jnp.zeros((NC_PAD, H), jnp.float32)
    zf = jnp.zeros((NF_PAD, H), jnp.float32)
    zc16 = jnp.zeros((NC_PAD, 16), jnp.float32)
    ones16 = jnp.concatenate(
        [jnp.ones((CHUNK, 1), jnp.float32), jnp.zeros((CHUNK, 15), jnp.float32)], axis=1)

    hf = _init_f(embed_feat, in_lin_W[1], in_lin_b[1], in_norm_g[1], in_norm_b[1])
    hc = _init_c(cell_ids, embed_cell, in_lin_W[0], in_lin_b[0], in_norm_g[0], in_norm_b[0])

    sck = _sc_kernels()
    dc16, df116, df216 = sck["deg"](f2c_d, c2f_d, pw_d, ones16, zc16)
    rc = _recip_deg(dc16, NC)
    rf1 = _recip_deg(df116, NF)
    rf2 = _recip_deg(df216, NF)

    cn_g_l = cn_g.reshape(L, 3, H)
    cn_b_l = cn_b.reshape(L, 3, H)

    for i in range(L):
        Ws, Wn, b = sage_Ws[i], sage_Wn[i], sage_b[i]
        g_l, b_l = cn_g_l[i], cn_b_l[i]
        s_c, s_f1, s_f2 = sck["seg_all"](hf, hc,
                                         f2c_s, f2c_d, f2c_wp,
                                         c2f_s, c2f_d, c2f_wp,
                                         pw_s, pw_d, pw_wp, zc)
        hc_new = _cell_dense(hc, s_c[:, :NC], rc, Ws[0], Wn[0],
                             b[0], g_l[0], b_l[0])
        hf_new = _feat_dense(hf, s_f1[:, :NF], rf1, s_f2[:, :NF], rf2,
                             Ws[1], Wn[1], b[1],
                             Ws[2], Wn[2], b[2],
                             g_l[1], b_l[1], g_l[2], b_l[2])
        hf, hc = hf_new, hc_new

    return jnp.concatenate([hf, hc], axis=0)


# speedup vs baseline: 1.5308x; 1.1336x over previous
import functools

import jax
import jax.numpy as jnp
from jax import lax
from jax.experimental import pallas as pl
from jax.experimental.pallas import tpu as pltpu
from jax.experimental.pallas import tpu_sc as plsc

NF = 4000
NC = 10000
H = 128
L = 2
G = 4

NCORE = 2
NSUB = 16
CHUNK = 128
KB = 16

NC_PAD = 10112
NF_PAD = 4096
E1_PAD = 327680
E2_PAD = 65536
CPW1_C0 = 144
CPW1_C1 = 16
CPW2_C0 = 32
CPW2_C1 = 0


def _scale_chunk(g, w_v, gbuf):
    def outer(i16, c):
        wv = w_v[g, pl.ds(i16 * 16, 16)]
        for e in range(16):
            s = wv[e]
            i = i16 * 16 + e
            for j in range(H // 16):
                gbuf[i, pl.ds(j * 16, 16)] = gbuf[i, pl.ds(j * 16, 16)] * s
        return c

    lax.fori_loop(0, CHUNK // 16, outer, 0, unroll=False)


def _seg_pipeline(h_hbm, src_hbm, dst_hbm, w_hbm, acc,
                  src_v, dst_v, w_v, bufs, sems, start_chunk, cpw):
    b0, b1 = bufs
    s0, s1 = sems
    nb = cpw // KB

    def start_gather(g, gbuf, sem):
        pltpu.async_copy(h_hbm.at[src_v.at[g]], gbuf, sem)

    def wait_gather(gbuf, sem):
        pltpu.make_async_copy(h_hbm.at[src_v.at[0]], gbuf, sem).wait()

    def block_body(b, carry):
        pltpu.sync_copy(src_hbm.at[pl.ds(start_chunk + b * KB, KB)], src_v)
        pltpu.sync_copy(dst_hbm.at[pl.ds(start_chunk + b * KB, KB)], dst_v)
        pltpu.sync_copy(w_hbm.at[pl.ds(start_chunk + b * KB, KB)], w_v)
        start_gather(0, b0, s0)
        start_gather(1, b1, s1)

        def pair_body(k, c2):
            g = 2 * k
            wait_gather(b0, s0)
            _scale_chunk(g, w_v, b0)
            pltpu.sync_copy(b0, acc.at[dst_v.at[g]], add=True)
            start_gather(g + 2, b0, s0)
            wait_gather(b1, s1)
            _scale_chunk(g + 1, w_v, b1)
            pltpu.sync_copy(b1, acc.at[dst_v.at[g + 1]], add=True)
            start_gather(g + 3, b1, s1)
            return c2

        lax.fori_loop(0, KB // 2 - 1, pair_body, 0, unroll=False)
        g = KB - 2
        wait_gather(b0, s0)
        _scale_chunk(g, w_v, b0)
        pltpu.sync_copy(b0, acc.at[dst_v.at[g]], add=True)
        wait_gather(b1, s1)
        _scale_chunk(g + 1, w_v, b1)
        pltpu.sync_copy(b1, acc.at[dst_v.at[g + 1]], add=True)
        return carry

    lax.fori_loop(0, nb, block_body, 0, unroll=False)


_SEG_SCRATCH = [
    pltpu.VMEM((KB, CHUNK), jnp.int32),
    pltpu.VMEM((KB, CHUNK), jnp.int32),
    pltpu.VMEM((KB, CHUNK), jnp.float32),
    pltpu.VMEM((CHUNK, H), jnp.float32),
    pltpu.VMEM((CHUNK, H), jnp.float32),
    pltpu.VMEM((8, 16), jnp.float32),
    pltpu.SemaphoreType.DMA,
    pltpu.SemaphoreType.DMA,
]


def _make_seg_all():
    cpw1 = E1_PAD // (CHUNK * NCORE * NSUB)
    cpw2 = E2_PAD // (CHUNK * NCORE * NSUB)
    rps_c = NC_PAD // NSUB
    rps_f = NF_PAD // NSUB
    mesh = plsc.VectorSubcoreMesh(
        core_axis_name="c", subcore_axis_name="s",
        num_cores=NCORE, num_subcores=NSUB)

    @functools.partial(
        pl.kernel,
        out_type=(jax.ShapeDtypeStruct((NCORE, NC_PAD, H), jnp.float32),
                  jax.ShapeDtypeStruct((NCORE, NF_PAD, H), jnp.float32),
                  jax.ShapeDtypeStruct((NCORE, NF_PAD, H), jnp.float32)),
        mesh=mesh,
        scratch_types=_SEG_SCRATCH[:5]
        + [pltpu.VMEM_SHARED((NC_PAD, H), jnp.float32)]
        + _SEG_SCRATCH[5:],
    )
    def seg3(hf_hbm, hc_hbm,
             src1_hbm, dst1_hbm, w1_hbm,
             src2_hbm, dst2_hbm, w2_hbm,
             src3_hbm, dst3_hbm, w3_hbm, zeros_hbm,
             outc_hbm, outf1_hbm, outf2_hbm,
             src_v, dst_v, w_v, b0, b1, acc, tok_v, s0, s1):
        cid = lax.axis_index("c")
        sid = lax.axis_index("s")
        wid = sid * NCORE + cid

        def zero_acc(rps):
            pltpu.sync_copy(zeros_hbm.at[pl.ds(sid * rps, rps)],
                            acc.at[pl.ds(sid * rps, rps)])

        def writeout(out_hbm, rps):
            pltpu.sync_copy(acc.at[pl.ds(sid * rps, rps)],
                            out_hbm.at[cid, pl.ds(sid * rps, rps)])

        def pipeline(h_hbm, src_hbm, dst_hbm, w_hbm, cpw0, cpw1c):
            if cpw0 > 0:
                @pl.when(cid == 0)
                def _():
                    _seg_pipeline(h_hbm, src_hbm, dst_hbm, w_hbm, acc,
                                  src_v, dst_v, w_v, (b0, b1), (s0, s1),
                                  sid * cpw0, cpw0)
            if cpw1c > 0:
                @pl.when(cid == 1)
                def _():
                    _seg_pipeline(h_hbm, src_hbm, dst_hbm, w_hbm, acc,
                                  src_v, dst_v, w_v, (b0, b1), (s0, s1),
                                  NSUB * cpw0 + sid * cpw1c, cpw1c)

        zero_acc(rps_c)
        plsc.subcore_barrier()
        pipeline(hf_hbm, src1_hbm, dst1_hbm, w1_hbm, CPW1_C0, CPW1_C1)
        plsc.subcore_barrier()
        writeout(outc_hbm, rps_c)
        plsc.subcore_barrier()
        zero_acc(rps_f)
        plsc.subcore_barrier()
        pipeline(hc_hbm, src2_hbm, dst2_hbm, w2_hbm, CPW1_C0, CPW1_C1)
        plsc.subcore_barrier()
        writeout(outf1_hbm, rps_f)
        plsc.subcore_barrier()
        zero_acc(rps_f)
        plsc.subcore_barrier()
        pipeline(hf_hbm, src3_hbm, dst3_hbm, w3_hbm, CPW2_C0, CPW2_C1)
        plsc.subcore_barrier()
        writeout(outf2_hbm, rps_f)

    return seg3


def _make_deg():
    cpw1 = E1_PAD // (CHUNK * NCORE * NSUB)
    cpw2 = E2_PAD // (CHUNK * NCORE * NSUB)
    rps_c = NC_PAD // NSUB
    rps_f = NF_PAD // NSUB
    mesh = plsc.VectorSubcoreMesh(
        core_axis_name="c", subcore_axis_name="s",
        num_cores=NCORE, num_subcores=NSUB)

    @functools.partial(
        pl.kernel,
        out_type=(jax.ShapeDtypeStruct((NCORE, NC_PAD, 16), jnp.float32),
                  jax.ShapeDtypeStruct((NCORE, NF_PAD, 16), jnp.float32),
                  jax.ShapeDtypeStruct((NCORE, NF_PAD, 16), jnp.float32)),
        mesh=mesh,
        scratch_types=[
            pltpu.VMEM((cpw1, CHUNK), jnp.int32),
            pltpu.VMEM((CHUNK, 16), jnp.float32),
            pltpu.VMEM_SHARED((NC_PAD, 16), jnp.float32),
        ],
    )
    def deg(dst1_hbm, dst2_hbm, dst3_hbm, ones_hbm, zeros_hbm,
            out1_hbm, out2_hbm, out3_hbm,
            dst_v, ones_v, acc):
        cid = lax.axis_index("c")
        sid = lax.axis_index("s")
        wid = sid * NCORE + cid

        pltpu.sync_copy(ones_hbm, ones_v)

        def zero_acc(rps):
            pltpu.sync_copy(zeros_hbm.at[pl.ds(sid * rps, rps)],
                            acc.at[pl.ds(sid * rps, rps)])

        def run_pass(dst_hbm, cpw):
            pltpu.sync_copy(dst_hbm.at[pl.ds(wid * cpw, cpw)],
                            dst_v.at[pl.ds(0, cpw)])

            def chunk(g, carry):
                pltpu.sync_copy(ones_v, acc.at[dst_v.at[g]], add=True)
                return carry

            lax.fori_loop(0, cpw, chunk, 0, unroll=False)

        def writeout(out_hbm, rps):
            pltpu.sync_copy(acc.at[pl.ds(sid * rps, rps)],
                            out_hbm.at[cid, pl.ds(sid * rps, rps)])

        zero_acc(rps_c)
        plsc.subcore_barrier()
        run_pass(dst1_hbm, cpw1)
        plsc.subcore_barrier()
        writeout(out1_hbm, rps_c)
        plsc.subcore_barrier()
        zero_acc(rps_f)
        plsc.subcore_barrier()
        run_pass(dst2_hbm, cpw1)
        plsc.subcore_barrier()
        writeout(out2_hbm, rps_f)
        plsc.subcore_barrier()
        zero_acc(rps_f)
        plsc.subcore_barrier()
        run_pass(dst3_hbm, cpw2)
        plsc.subcore_barrier()
        writeout(out3_hbm, rps_f)

    return deg


def _gelu(x):
    return 0.5 * x * (1.0 + lax.erf(x * 0.7071067811865476))


def _group_masks():
    gpc = H // G
    Mg = (lax.broadcasted_iota(jnp.int32, (H, G), 0) // gpc
          == lax.broadcasted_iota(jnp.int32, (H, G), 1)).astype(jnp.float32)
    MgT = (lax.broadcasted_iota(jnp.int32, (G, H), 1) // gpc
           == lax.broadcasted_iota(jnp.int32, (G, H), 0)).astype(jnp.float32)
    return Mg / gpc, MgT


def _gn_apply(x, g, b):
    Mg, MgT = _group_masks()
    m = lax.dot(x, Mg, preferred_element_type=jnp.float32)
    mb = lax.dot(m, MgT, preferred_element_type=jnp.float32)
    xc = x - mb
    v = lax.dot(xc * xc, Mg, preferred_element_type=jnp.float32)
    vb = lax.dot(v, MgT, preferred_element_type=jnp.float32)
    return xc * lax.rsqrt(vb + 1e-5) * g + b


def _dot(a, b):
    return lax.dot(a, b, preferred_element_type=jnp.float32)


def _init_f_body(x_ref, W_ref, b_ref, g_ref, bn_ref, o_ref):
    x = x_ref[...]
    x = jnp.where(x >= 0, x, 0.01 * x)
    y = _dot(x, W_ref[...]) + b_ref[...]
    o_ref[...] = _gn_apply(_gelu(y), g_ref[...], bn_ref[...])


def _init_c_body(ids_ref, ec_ref, W_ref, b_ref, g_ref, bn_ref, o_ref):
    ids = ids_ref[...]
    ec = ec_ref[...]
    x = jnp.where(ids == 0, ec[0:1, :], ec[1:2, :])
    x = jnp.where(x >= 0, x, 0.01 * x)
    y = _dot(x, W_ref[...]) + b_ref[...]
    o_ref[...] = _gn_apply(_gelu(y), g_ref[...], bn_ref[...])


def _cell_body(hc_ref, s_ref, r_ref, Ws_ref, Wn_ref, b_ref, g_ref, bn_ref, o_ref):
    s = (s_ref[0] + s_ref[1]) * r_ref[...]
    m = _dot(hc_ref[...], Ws_ref[...]) + _dot(s, Wn_ref[...]) + b_ref[...]
    o_ref[...] = _gelu(_gn_apply(m, g_ref[...], bn_ref[...]))


def _feat_body(hf_ref, s1_ref, r1_ref, s2_ref, r2_ref,
               Ws1_ref, Wn1_ref, b1_ref, Ws2_ref, Wn2_ref, b2_ref,
               g1_ref, bn1_ref, g2_ref, bn2_ref, o_ref):
    hf = hf_ref[...]
    s1 = (s1_ref[0] + s1_ref[1]) * r1_ref[...]
    s2 = (s2_ref[0] + s2_ref[1]) * r2_ref[...]
    m1 = _dot(hf, Ws1_ref[...]) + _dot(s1, Wn1_ref[...]) + b1_ref[...]
    m2 = _dot(hf, Ws2_ref[...]) + _dot(s2, Wn2_ref[...]) + b2_ref[...]
    h1 = _gn_apply(m1, g1_ref[...], bn1_ref[...])
    h2 = _gn_apply(m2, g2_ref[...], bn2_ref[...])
    o_ref[...] = _gelu(0.5 * h1 + 0.5 * h2)


def _row_spec(bn):
    return pl.BlockSpec((bn, H), lambda i: (i, 0))


def _full_spec(shape):
    return pl.BlockSpec(shape, lambda i: tuple(0 for _ in shape))


def _init_f(x, W, b, g, bn):
    bn_rows = 1000
    return pl.pallas_call(
        _init_f_body,
        grid=(NF // bn_rows,),
        in_specs=[_row_spec(bn_rows), _full_spec((H, H)), _full_spec((1, H)),
                  _full_spec((1, H)), _full_spec((1, H))],
        out_specs=_row_spec(bn_rows),
        out_shape=jax.ShapeDtypeStruct((NF, H), jnp.float32),
    )(x, W, b.reshape(1, H), g.reshape(1, H), bn.reshape(1, H))


def _init_c(ids, ec, W, b, g, bn):
    bn_rows = 1000
    return pl.pallas_call(
        _init_c_body,
        grid=(NC // bn_rows,),
        in_specs=[pl.BlockSpec((bn_rows, 1), lambda i: (i, 0)),
                  _full_spec((2, H)), _full_spec((H, H)), _full_spec((1, H)),
                  _full_spec((1, H)), _full_spec((1, H))],
        out_specs=_row_spec(bn_rows),
        out_shape=jax.ShapeDtypeStruct((NC, H), jnp.float32),
    )(ids.reshape(NC, 1), ec, W, b.reshape(1, H), g.reshape(1, H), bn.reshape(1, H))


def _cell_dense(hc, s, r, Ws, Wn, b, g, bn):
    bn_rows = 1000
    return pl.pallas_call(
        _cell_body,
        grid=(NC // bn_rows,),
        in_specs=[_row_spec(bn_rows),
                  pl.BlockSpec((NCORE, bn_rows, H), lambda i: (0, i, 0)),
                  pl.BlockSpec((bn_rows, 1), lambda i: (i, 0)),
                  _full_spec((H, H)), _full_spec((H, H)), _full_spec((1, H)),
                  _full_spec((1, H)), _full_spec((1, H))],
        out_specs=_row_spec(bn_rows),
        out_shape=jax.ShapeDtypeStruct((NC, H), jnp.float32),
    )(hc, s, r, Ws, Wn, b.reshape(1, H), g.reshape(1, H), bn.reshape(1, H))


def _feat_dense(hf, s1, r1, s2, r2, Ws1, Wn1, b1, Ws2, Wn2, b2, g1, bn1, g2, bn2):
    bn_rows = 1000
    return pl.pallas_call(
        _feat_body,
        grid=(NF // bn_rows,),
        in_specs=[_row_spec(bn_rows),
                  pl.BlockSpec((NCORE, bn_rows, H), lambda i: (0, i, 0)),
                  pl.BlockSpec((bn_rows, 1), lambda i: (i, 0)),
                  pl.BlockSpec((NCORE, bn_rows, H), lambda i: (0, i, 0)),
                  pl.BlockSpec((bn_rows, 1), lambda i: (i, 0)),
                  _full_spec((H, H)), _full_spec((H, H)), _full_spec((1, H)),
                  _full_spec((H, H)), _full_spec((H, H)), _full_spec((1, H)),
                  _full_spec((1, H)), _full_spec((1, H)),
                  _full_spec((1, H)), _full_spec((1, H))],
        out_specs=_row_spec(bn_rows),
        out_shape=jax.ShapeDtypeStruct((NF, H), jnp.float32),
    )(hf, s1, r1, s2, r2, Ws1, Wn1, b1.reshape(1, H), Ws2, Wn2, b2.reshape(1, H),
      g1.reshape(1, H), bn1.reshape(1, H), g2.reshape(1, H), bn2.reshape(1, H))


@functools.lru_cache(None)
def _sc_kernels():
    return {
        "seg_all": _make_seg_all(),
        "deg": _make_deg(),
    }


def _pad_edges(src, dst, w, e_pad, n_dst):
    e = src.shape[0]
    pad = e_pad - e
    shp = (e_pad // CHUNK, CHUNK)
    src_p = jnp.concatenate([src, jnp.zeros((pad,), jnp.int32)]).reshape(shp)
    dst_p = jnp.concatenate([dst, jnp.full((pad,), n_dst, jnp.int32)]).reshape(shp)
    w_p = jnp.concatenate([w, jnp.zeros((pad,), jnp.float32)]).reshape(shp)
    return src_p, dst_p, w_p


def _recip_deg(deg16, n):
    d = deg16[:, :n, 0].sum(0)
    return (1.0 / jnp.clip(d, 1.0)).reshape(n, 1)


def kernel(feat_ids, cell_ids, f2c_src, f2c_dst, f2c_w, c2f_src, c2f_dst, c2f_w,
           pw_src, pw_dst, pw_w, embed_feat, embed_cell, in_lin_W, in_lin_b,
           in_norm_g, in_norm_b, sage_Wn, sage_Ws, sage_b, cn_g, cn_b):
    del feat_ids

    f2c_s, f2c_d, f2c_wp = _pad_edges(f2c_src, f2c_dst, f2c_w, E1_PAD, NC)
    c2f_s, c2f_d, c2f_wp = _pad_edges(c2f_src, c2f_dst, c2f_w, E1_PAD, NF)
    pw_s, pw_d, pw_wp = _pad_edges(pw_src, pw_dst, pw_w, E2_PAD, NF)

    zc = jnp.zeros((NC_PAD, H), jnp.float32)
    zf = jnp.zeros((NF_PAD, H), jnp.float32)
    zc16 = jnp.zeros((NC_PAD, 16), jnp.float32)
    ones16 = jnp.concatenate(
        [jnp.ones((CHUNK, 1), jnp.float32), jnp.zeros((CHUNK, 15), jnp.float32)], axis=1)

    hf = _init_f(embed_feat, in_lin_W[1], in_lin_b[1], in_norm_g[1], in_norm_b[1])
    hc = _init_c(cell_ids, embed_cell, in_lin_W[0], in_lin_b[0], in_norm_g[0], in_norm_b[0])

    sck = _sc_kernels()
    dc16, df116, df216 = sck["deg"](f2c_d, c2f_d, pw_d, ones16, zc16)
    rc = _recip_deg(dc16, NC)
    rf1 = _recip_deg(df116, NF)
    rf2 = _recip_deg(df216, NF)

    cn_g_l = cn_g.reshape(L, 3, H)
    cn_b_l = cn_b.reshape(L, 3, H)

    for i in range(L):
        Ws, Wn, b = sage_Ws[i], sage_Wn[i], sage_b[i]
        g_l, b_l = cn_g_l[i], cn_b_l[i]
        s_c, s_f1, s_f2 = sck["seg_all"](hf, hc,
                                         f2c_s, f2c_d, f2c_wp,
                                         c2f_s, c2f_d, c2f_wp,
                                         pw_s, pw_d, pw_wp, zc)
        hc_new = _cell_dense(hc, s_c[:, :NC], rc, Ws[0], Wn[0],
                             b[0], g_l[0], b_l[0])
        hf_new = _feat_dense(hf, s_f1[:, :NF], rf1, s_f2[:, :NF], rf2,
                             Ws[1], Wn[1], b[1],
                             Ws[2], Wn[2], b[2],
                             g_l[1], b_l[1], g_l[2], b_l[2])
        hf, hc = hf_new, hc_new

    return jnp.concatenate([hf, hc], axis=0)
